# baseline (speedup 1.0000x reference)
import functools

import jax
import jax.numpy as jnp
from jax import lax
from jax.experimental import pallas as pl
from jax.experimental.pallas import tpu as pltpu
from jax.experimental.pallas import tpu_sc as plsc

CODES = 1024
SPECIAL = 2
QUANT = 8
WIDTH = 384
LENGTH = 2250
BATCH = 32
ROWS = CODES + SPECIAL

CHUNK = 128
PAD_N = 2304
NCHUNK = PAD_N // CHUNK
SUB = 32
NSUB = CHUNK // SUB
TAIL = LENGTH - (NCHUNK - 1) * CHUNK
TAIL_FULL = TAIL // SUB
TAIL_REST = TAIL - TAIL_FULL * SUB
LANES = 16


def _make_embed_sum():
    mesh = plsc.VectorSubcoreMesh(core_axis_name="c", subcore_axis_name="s")
    num_cores = mesh.num_cores

    @functools.partial(
        pl.kernel,
        out_type=jax.ShapeDtypeStruct((BATCH, LENGTH, WIDTH), jnp.float32),
        mesh=mesh,
        scratch_types=[
            pltpu.VMEM((QUANT, LENGTH), jnp.int32),
            pltpu.VMEM((QUANT, PAD_N), jnp.int32),
            pltpu.VMEM((2, CHUNK, WIDTH), jnp.bfloat16),
            pltpu.VMEM((2, SUB, WIDTH), jnp.float32),
            pltpu.SemaphoreType.DMA,
            pltpu.SemaphoreType.DMA,
        ],
        compiler_params=pltpu.CompilerParams(
            use_tc_tiling_on_sc=False, needs_layout_passes=False
        ),
    )
    def embed_sum(
        toks_hbm, table_hbm, out_hbm, tok_v, idx_v, acc, fbuf, sem_g, sem_o
    ):
        wid = lax.axis_index("s") * num_cores + lax.axis_index("c")
        iota = lax.iota(jnp.int32, LANES)

        pltpu.sync_copy(toks_hbm.at[wid], tok_v)

        def fix_body(g, carry):
            sl = pl.ds(g * LANES, LANES)
            for q in range(QUANT):
                t = tok_v[q, sl]
                idx_v[q, sl] = jnp.clip(t, 0, ROWS - 1) + q * ROWS
            return carry

        lax.fori_loop(0, LENGTH // LANES, fix_body, 0)

        last = (LENGTH // LANES) * LANES
        maxcol = jnp.full((LANES,), LENGTH - 1, jnp.int32)
        for q in range(QUANT):
            qvec = jnp.full((LANES,), q, jnp.int32)
            t = plsc.load_gather(tok_v, [qvec, jnp.minimum(last + iota, maxcol)])
            idx_v[q, pl.ds(last, LANES)] = jnp.clip(t, 0, ROWS - 1) + q * ROWS
            for g in range(last // LANES + 1, PAD_N // LANES):
                idx_v[q, pl.ds(g * LANES, LANES)] = jnp.zeros((LANES,), jnp.int32)

        zeros_bf = jnp.zeros((2 * LANES,), jnp.bfloat16)
        iota2 = 2 * iota
        mask_hi = jnp.full((LANES,), -65536, jnp.int32)

        def convert_sub(par, base, s, rows):
            fpar = s % 2

            def row_body(p, carry2):
                for j in range(WIDTH // (2 * LANES)):
                    w = plsc.bitcast(
                        acc[par, s * SUB + p, pl.ds(j * 2 * LANES, 2 * LANES)],
                        jnp.int32,
                    )
                    even = plsc.bitcast(lax.shift_left(w, 16), jnp.float32)
                    odd = plsc.bitcast(
                        lax.bitwise_and(w, mask_hi), jnp.float32
                    )
                    cols = iota2 + (j * 2 * LANES)
                    pvec = jnp.full((LANES,), p, jnp.int32)
                    fvec = jnp.full((LANES,), fpar, jnp.int32)
                    plsc.store_scatter(fbuf, [fvec, pvec, cols], even)
                    plsc.store_scatter(fbuf, [fvec, pvec, cols + 1], odd)
                return carry2

            lax.fori_loop(0, rows, row_body, 0)
            pltpu.async_copy(
                fbuf.at[fpar, pl.ds(0, rows)],
                out_hbm.at[wid, pl.ds(base + s * SUB, rows)],
                sem_o,
            )

        def drain_out(rows):
            pltpu.make_async_copy(
                fbuf.at[0, pl.ds(0, rows)],
                out_hbm.at[wid, pl.ds(0, rows)],
                sem_o,
            ).wait()

        def zero_par(par):
            def zero_body(p, carry2):
                for j in range(WIDTH // (2 * LANES)):
                    acc[par, p, pl.ds(j * 2 * LANES, 2 * LANES)] = zeros_bf
                return carry2

            lax.fori_loop(0, CHUNK, zero_body, 0)

        zero_par(0)

        def chunk_body(c, carry):
            base = pl.multiple_of(c * CHUNK, CHUNK)
            par = lax.rem(c, 2)
            ppar = lax.rem(c + 1, 2)

            descs = [
                pltpu.async_copy(
                    table_hbm.at[idx_v.at[q, pl.ds(base, CHUNK)]],
                    acc.at[par],
                    sem_g,
                    add=True,
                )
                for q in range(QUANT)
            ]

            @pl.when(c >= 1)
            def _():
                prev_base = base - CHUNK
                for s in range(NSUB):
                    if s < 2:
                        @pl.when(c >= 2)
                        def _():
                            drain_out(SUB)
                    else:
                        drain_out(SUB)
                    convert_sub(ppar, prev_base, s, SUB)

            zero_par(ppar)

            for d in descs:
                d.wait()

            return carry

        lax.fori_loop(0, NCHUNK, chunk_body, 0)

        last_par = (NCHUNK - 1) % 2
        last_base = (NCHUNK - 1) * CHUNK
        for s in range(TAIL_FULL):
            drain_out(SUB)
            convert_sub(last_par, last_base, s, SUB)
        drain_out(SUB)
        convert_sub(last_par, last_base, TAIL_FULL, TAIL_REST)

        drain_out(SUB)
        drain_out(TAIL_REST)

    return embed_sum


def kernel(toks, xenc, mains, special):
    del xenc
    toks32 = toks.astype(jnp.int32)
    table = (
        jnp.concatenate(
            [mains, jnp.broadcast_to(special[None], (QUANT, SPECIAL, WIDTH))],
            axis=1,
        )
        .reshape(QUANT * ROWS, WIDTH)
        .astype(jnp.bfloat16)
    )
    return _make_embed_sum()(toks32, table)


# speedup vs baseline: 1.5503x; 1.5503x over previous
import functools

import jax
import jax.numpy as jnp
from jax import lax
from jax.experimental import pallas as pl
from jax.experimental.pallas import tpu as pltpu
from jax.experimental.pallas import tpu_sc as plsc

CODES = 1024
SPECIAL = 2
QUANT = 8
WIDTH = 384
LENGTH = 2250
BATCH = 32
ROWS = CODES + SPECIAL

CHUNK = 128
PAD_N = 2304
NCHUNK = PAD_N // CHUNK
SUB = 32
NSUB = CHUNK // SUB
TAIL = LENGTH - (NCHUNK - 1) * CHUNK
TAIL_FULL = TAIL // SUB
TAIL_REST = TAIL - TAIL_FULL * SUB
LANES = 16


def _make_embed_sum():
    mesh = plsc.VectorSubcoreMesh(core_axis_name="c", subcore_axis_name="s")
    num_cores = mesh.num_cores

    @functools.partial(
        pl.kernel,
        out_type=jax.ShapeDtypeStruct((BATCH, LENGTH, WIDTH), jnp.float32),
        mesh=mesh,
        scratch_types=[
            pltpu.VMEM((QUANT, PAD_N), jnp.int32),
            pltpu.VMEM((2, CHUNK, WIDTH), jnp.bfloat16),
            pltpu.VMEM((2, SUB, WIDTH), jnp.float32),
            pltpu.SemaphoreType.DMA,
            pltpu.SemaphoreType.DMA,
            pltpu.SemaphoreType.DMA,
        ],
        compiler_params=pltpu.CompilerParams(
            use_tc_tiling_on_sc=False, needs_layout_passes=False
        ),
    )
    def embed_sum(
        toks_hbm, table_hbm, out_hbm, idx_v, acc, fbuf, sem_ga, sem_gb, sem_o
    ):
        wid = lax.axis_index("s") * num_cores + lax.axis_index("c")

        pltpu.sync_copy(toks_hbm.at[wid], idx_v)

        def fix_body(g, carry):
            sl = pl.ds(g * LANES, LANES)
            for q in range(QUANT):
                t = idx_v[q, sl]
                idx_v[q, sl] = jnp.clip(t, 0, ROWS - 1) + q * ROWS
            return carry

        lax.fori_loop(0, PAD_N // LANES, fix_body, 0)

        zeros_bf = jnp.zeros((2 * LANES,), jnp.bfloat16)
        iota2 = 2 * jax.lax.iota(jnp.int32, LANES)
        mask_hi = jnp.full((LANES,), -65536, jnp.int32)

        def convert_sub(par, base, s, rows):
            fpar = s % 2

            def row_body(p, carry2):
                for j in range(WIDTH // (2 * LANES)):
                    w = plsc.bitcast(
                        acc[par, s * SUB + p, pl.ds(j * 2 * LANES, 2 * LANES)],
                        jnp.int32,
                    )
                    even = plsc.bitcast(lax.shift_left(w, 16), jnp.float32)
                    odd = plsc.bitcast(
                        lax.bitwise_and(w, mask_hi), jnp.float32
                    )
                    cols = iota2 + (j * 2 * LANES)
                    pvec = jnp.full((LANES,), p, jnp.int32)
                    fvec = jnp.full((LANES,), fpar, jnp.int32)
                    plsc.store_scatter(fbuf, [fvec, pvec, cols], even)
                    plsc.store_scatter(fbuf, [fvec, pvec, cols + 1], odd)
                return carry2

            lax.fori_loop(0, rows, row_body, 0)
            pltpu.async_copy(
                fbuf.at[fpar, pl.ds(0, rows)],
                out_hbm.at[wid, pl.ds(base + s * SUB, rows)],
                sem_o,
            )

        def drain_out(rows):
            pltpu.make_async_copy(
                fbuf.at[0, pl.ds(0, rows)],
                out_hbm.at[wid, pl.ds(0, rows)],
                sem_o,
            ).wait()

        def zero_par(par):
            def zero_body(p, carry2):
                for j in range(WIDTH // (2 * LANES)):
                    acc[par, p, pl.ds(j * 2 * LANES, 2 * LANES)] = zeros_bf
                return carry2

            lax.fori_loop(0, CHUNK, zero_body, 0)

        def issue_gathers(c, par, sem):
            base = pl.multiple_of(c * CHUNK, CHUNK)
            for q in range(QUANT):
                pltpu.async_copy(
                    table_hbm.at[idx_v.at[q, pl.ds(base, CHUNK)]],
                    acc.at[par],
                    sem,
                    add=True,
                )

        def wait_gathers(sem):
            for _ in range(QUANT):
                pltpu.make_async_copy(
                    table_hbm.at[idx_v.at[0, pl.ds(0, CHUNK)]],
                    acc.at[0],
                    sem,
                ).wait()

        zero_par(0)
        zero_par(1)
        issue_gathers(0, 0, sem_ga)

        def pair_body(k, carry):
            @pl.when(k >= 1)
            def _():
                for s in range(NSUB):
                    drain_out(SUB)
                    convert_sub(1, (2 * k - 1) * CHUNK, s, SUB)
                zero_par(1)

            issue_gathers(2 * k + 1, 1, sem_gb)
            wait_gathers(sem_ga)

            for s in range(NSUB):
                if s < 2:
                    @pl.when(k >= 1)
                    def _():
                        drain_out(SUB)
                else:
                    drain_out(SUB)
                convert_sub(0, 2 * k * CHUNK, s, SUB)
            zero_par(0)

            @pl.when(k < NCHUNK // 2 - 1)
            def _():
                issue_gathers(2 * k + 2, 0, sem_ga)

            wait_gathers(sem_gb)
            return carry

        lax.fori_loop(0, NCHUNK // 2, pair_body, 0)

        last_par = (NCHUNK - 1) % 2
        last_base = (NCHUNK - 1) * CHUNK
        for s in range(TAIL_FULL):
            drain_out(SUB)
            convert_sub(last_par, last_base, s, SUB)
        drain_out(SUB)
        convert_sub(last_par, last_base, TAIL_FULL, TAIL_REST)

        drain_out(SUB)
        drain_out(TAIL_REST)

    return embed_sum


def kernel(toks, xenc, mains, special):
    del xenc
    toks32 = jnp.pad(
        toks.astype(jnp.int32), ((0, 0), (0, 0), (0, PAD_N - LENGTH))
    )
    table = (
        jnp.concatenate(
            [mains, jnp.broadcast_to(special[None], (QUANT, SPECIAL, WIDTH))],
            axis=1,
        )
        .reshape(QUANT * ROWS, WIDTH)
        .astype(jnp.bfloat16)
    )
    return _make_embed_sum()(toks32, table)
